# Initial kernel scaffold; baseline (speedup 1.0000x reference)
import jax
import jax.numpy as jnp
from jax.experimental import pallas as pl


def kernel(h, A, padded_neighbor_list, gc1_W1_0, gc1_b1_0, gc1_W2_0, gc1_b2_0, gc1_gamma_0, gc1_beta_0, gc2_W1_0, gc2_b1_0, gc2_W2_0, gc2_b2_0, gc2_gamma_0, gc2_beta_0, gc1_W1_1, gc1_b1_1, gc1_W2_1, gc1_b2_1, gc1_gamma_1, gc1_beta_1, gc2_W1_1, gc2_b1_1, gc2_W2_1, gc2_b2_1, gc2_gamma_1, gc2_beta_1, gc1_W1_2, gc1_b1_2, gc1_W2_2, gc1_b2_2, gc1_gamma_2, gc1_beta_2, gc2_W1_2, gc2_b1_2, gc2_W2_2, gc2_b2_2, gc2_gamma_2, gc2_beta_2, gc1_W1_3, gc1_b1_3, gc1_W2_3, gc1_b2_3, gc1_gamma_3, gc1_beta_3, gc2_W1_3, gc2_b1_3, gc2_W2_3, gc2_b2_3, gc2_gamma_3, gc2_beta_3, gc1_W1_4, gc1_b1_4, gc1_W2_4, gc1_b2_4, gc1_gamma_4, gc1_beta_4, gc2_W1_4, gc2_b1_4, gc2_W2_4, gc2_b2_4, gc2_gamma_4, gc2_beta_4, Wc1, bc1, alpha, Wc2, bc2):
    raise NotImplementedError("write your pallas kernel here")


# speedup vs baseline: 2.5228x; 2.5228x over previous
import functools

import jax
import jax.numpy as jnp
from jax import lax
from jax.experimental import pallas as pl
from jax.experimental.pallas import tpu as pltpu
from jax.experimental.pallas import tpu_sc as plsc

_N = 2048
_DEG = 16
_EPS = 1e-5
_RB = 256
_KS = _N // _RB
_NC, _NS = 2, 16
_NW = _NC * _NS
_NPW = _N // _NW


def _pad128(d):
    return ((d + 127) // 128) * 128


@functools.lru_cache(maxsize=None)
def _make_conv(d, h1, h2):
    f32 = jnp.float32
    P = _pad128(d)

    def body(A_ref, X_ref,
             W1a_ref, b1a_ref, W2a_ref, b2a_ref, ga_ref, bta_ref,
             W1b_ref, b1b_ref, W2b_ref, b2b_ref, gb_ref, btb_ref,
             y1_ref, y2_ref, table_ref,
             y1s, y2s, s1, q1, s2, q2):
        j = pl.program_id(0)

        @pl.when(j == 0)
        def _():
            s1[...] = jnp.zeros_like(s1)
            q1[...] = jnp.zeros_like(q1)
            s2[...] = jnp.zeros_like(s2)
            q2[...] = jnp.zeros_like(q2)

        X = X_ref[...]
        agg = jnp.dot(A_ref[...], X, preferred_element_type=f32)
        t1 = jnp.maximum(
            jnp.dot(agg, W1a_ref[...], preferred_element_type=f32) + b1a_ref[...], 0.0)
        r1 = jnp.dot(t1, W2a_ref[...], preferred_element_type=f32) + b2a_ref[...]
        t2 = jnp.maximum(
            jnp.dot(agg, W1b_ref[...], preferred_element_type=f32) + b1b_ref[...], 0.0)
        r2 = jnp.dot(t2, W2b_ref[...], preferred_element_type=f32) + b2b_ref[...]
        y1s[pl.ds(j * _RB, _RB), :] = r1
        y2s[pl.ds(j * _RB, _RB), :] = r2
        s1[...] += jnp.sum(r1, axis=0, keepdims=True)
        q1[...] += jnp.sum(r1 * r1, axis=0, keepdims=True)
        s2[...] += jnp.sum(r2, axis=0, keepdims=True)
        q2[...] += jnp.sum(r2 * r2, axis=0, keepdims=True)
        Xb = X_ref[pl.ds(j * _RB, _RB), :]
        table_ref[pl.ds(j * _RB, _RB), :] = jnp.concatenate(
            [Xb, jnp.zeros((_RB, P - d), f32)], axis=1)

        @pl.when(j == _KS - 1)
        def _():
            inv_n = 1.0 / _N
            m1 = s1[...] * inv_n
            v1 = q1[...] * inv_n - m1 * m1
            sc1 = ga_ref[...] * lax.rsqrt(v1 + _EPS)
            y1_ref[...] = jnp.maximum((y1s[...] - m1) * sc1 + bta_ref[...], 0.0)
            m2 = s2[...] * inv_n
            v2 = q2[...] * inv_n - m2 * m2
            sc2 = gb_ref[...] * lax.rsqrt(v2 + _EPS)
            y2_ref[...] = jnp.maximum((y2s[...] - m2) * sc2 + btb_ref[...], 0.0)
            cmin = jnp.min(X, axis=0, keepdims=True)
            table_ref[pl.ds(_N, 1), :] = jnp.concatenate(
                [cmin, jnp.zeros((1, P - d), f32)], axis=1)

    def whole(shape):
        nd = len(shape)
        return pl.BlockSpec(shape, lambda j, _nd=nd: (0,) * _nd)

    return pl.pallas_call(
        body,
        grid=(_KS,),
        in_specs=[
            pl.BlockSpec((_RB, _N), lambda j: (j, 0)),
            whole((_N, d)),
            whole((d, h1)), whole((1, h1)), whole((h1, 128)), whole((1, 128)),
            whole((1, 128)), whole((1, 128)),
            whole((d, h2)), whole((1, h2)), whole((h2, 64)), whole((1, 64)),
            whole((1, 64)), whole((1, 64)),
        ],
        out_specs=[whole((_N, 128)), whole((_N, 64)), whole((_N + 1, P))],
        out_shape=[
            jax.ShapeDtypeStruct((_N, 128), f32),
            jax.ShapeDtypeStruct((_N, 64), f32),
            jax.ShapeDtypeStruct((_N + 1, P), f32),
        ],
        scratch_shapes=[
            pltpu.VMEM((_N, 128), f32), pltpu.VMEM((_N, 64), f32),
            pltpu.VMEM((1, 128), f32), pltpu.VMEM((1, 128), f32),
            pltpu.VMEM((1, 64), f32), pltpu.VMEM((1, 64), f32),
        ],
        compiler_params=pltpu.CompilerParams(
            dimension_semantics=("arbitrary",)),
    )


@functools.lru_cache(maxsize=None)
def _make_maxpool(d):
    f32 = jnp.float32
    P = _pad128(d)
    npc = 8 if (_DEG * P * 4 * 2 * 8) <= 420000 else 4
    gs = npc * _DEG
    nch = _NPW // npc
    cpw = 8 // npc
    mesh = plsc.VectorSubcoreMesh(core_axis_name="c", subcore_axis_name="s")

    @functools.partial(
        pl.kernel,
        mesh=mesh,
        out_type=jax.ShapeDtypeStruct((_N, P), f32),
        scratch_types=[
            pltpu.VMEM((gs, P), f32),
            pltpu.VMEM((gs, P), f32),
            pltpu.VMEM((gs,), jnp.int32),
            pltpu.VMEM((gs,), jnp.int32),
            pltpu.VMEM((8, P), f32),
            pltpu.SemaphoreType.DMA,
            pltpu.SemaphoreType.DMA,
        ],
    )
    def mp(table_hbm, pnl_hbm, out_hbm, rows0, rows1, idx0, idx1, obuf,
           sem0, sem1):
        wid = lax.axis_index("s") * _NC + lax.axis_index("c")
        node0 = wid * _NPW
        rows = (rows0, rows1)
        idxs = (idx0, idx1)
        sems = (sem0, sem1)

        def start(g, b):
            pltpu.sync_copy(
                pnl_hbm.at[pl.ds((node0 + g * npc) * _DEG, gs)], idxs[b])
            pltpu.async_copy(table_hbm.at[idxs[b]], rows[b], sems[b])

        start(0, 0)
        for g in range(nch):
            b = g % 2
            pltpu.make_async_copy(table_hbm.at[idxs[b]], rows[b],
                                  sems[b]).wait()
            if g + 1 < nch:
                start(g + 1, 1 - b)
            r = rows[b]
            orow = (g % cpw) * npc
            for p in range(npc):
                def col(ci, carry, _p=p, _orow=orow):
                    c0 = ci * 16
                    a = r[_p * _DEG, pl.ds(c0, 16)]
                    for k in range(1, _DEG):
                        a = jnp.maximum(a, r[_p * _DEG + k, pl.ds(c0, 16)])
                    obuf[_orow + _p, pl.ds(c0, 16)] = a
                    return carry
                lax.fori_loop(0, P // 16, col, 0)
            if (g + 1) % cpw == 0:
                pltpu.sync_copy(
                    obuf, out_hbm.at[pl.ds(node0 + (g + 1 - cpw) * npc, 8)])

    return mp


@functools.lru_cache(maxsize=None)
def _make_head(d):
    f32 = jnp.float32

    def body(X_ref, W1_ref, b1_ref, al_ref, W2_ref, b2_ref, out_ref):
        pooled = jnp.sum(X_ref[...], axis=0, keepdims=True) * (1.0 / _N)
        z = jnp.dot(pooled, W1_ref[...], preferred_element_type=f32) + b1_ref[...]
        z = jnp.where(z > 0.0, z, al_ref[...] * z)
        r = jnp.dot(z, W2_ref[...], preferred_element_type=f32) + b2_ref[...]
        rp = jnp.concatenate([r, jnp.zeros((1, 126), f32)], axis=1)
        out_ref[...] = jnp.concatenate([rp, jnp.zeros((7, 128), f32)], axis=0)

    return pl.pallas_call(
        body,
        out_shape=jax.ShapeDtypeStruct((8, 128), f32),
    )


def kernel(h, A, padded_neighbor_list,
           gc1_W1_0, gc1_b1_0, gc1_W2_0, gc1_b2_0, gc1_gamma_0, gc1_beta_0,
           gc2_W1_0, gc2_b1_0, gc2_W2_0, gc2_b2_0, gc2_gamma_0, gc2_beta_0,
           gc1_W1_1, gc1_b1_1, gc1_W2_1, gc1_b2_1, gc1_gamma_1, gc1_beta_1,
           gc2_W1_1, gc2_b1_1, gc2_W2_1, gc2_b2_1, gc2_gamma_1, gc2_beta_1,
           gc1_W1_2, gc1_b1_2, gc1_W2_2, gc1_b2_2, gc1_gamma_2, gc1_beta_2,
           gc2_W1_2, gc2_b1_2, gc2_W2_2, gc2_b2_2, gc2_gamma_2, gc2_beta_2,
           gc1_W1_3, gc1_b1_3, gc1_W2_3, gc1_b2_3, gc1_gamma_3, gc1_beta_3,
           gc2_W1_3, gc2_b1_3, gc2_W2_3, gc2_b2_3, gc2_gamma_3, gc2_beta_3,
           gc1_W1_4, gc1_b1_4, gc1_W2_4, gc1_b2_4, gc1_gamma_4, gc1_beta_4,
           gc2_W1_4, gc2_b1_4, gc2_W2_4, gc2_b2_4, gc2_gamma_4, gc2_beta_4,
           Wc1, bc1, alpha, Wc2, bc2):
    prm = dict(locals())
    f32 = jnp.float32
    X = h[0].astype(f32)
    pnl_flat = padded_neighbor_list.astype(jnp.int32).reshape(-1)

    for i in range(5):
        d = X.shape[1]
        W1a = prm['gc1_W1_%d' % i]; h1 = W1a.shape[1]
        W1b = prm['gc2_W1_%d' % i]; h2 = W1b.shape[1]
        conv = _make_conv(d, h1, h2)
        y1, y2, table = conv(
            A, X,
            W1a, prm['gc1_b1_%d' % i].reshape(1, -1),
            prm['gc1_W2_%d' % i], prm['gc1_b2_%d' % i].reshape(1, -1),
            prm['gc1_gamma_%d' % i].reshape(1, -1),
            prm['gc1_beta_%d' % i].reshape(1, -1),
            W1b, prm['gc2_b1_%d' % i].reshape(1, -1),
            prm['gc2_W2_%d' % i], prm['gc2_b2_%d' % i].reshape(1, -1),
            prm['gc2_gamma_%d' % i].reshape(1, -1),
            prm['gc2_beta_%d' % i].reshape(1, -1),
        )
        o1 = _make_maxpool(d)(table, pnl_flat)
        X = jnp.concatenate([o1[:, :d], y1, y2], axis=1)

    head = _make_head(X.shape[1])
    res = head(X, Wc1, bc1.reshape(1, -1), alpha.reshape(1, -1),
               Wc2, bc2.reshape(1, -1))
    return res[0:1, 0:2]


# speedup vs baseline: 2.9932x; 1.1865x over previous
import functools

import jax
import jax.numpy as jnp
from jax import lax
from jax.experimental import pallas as pl
from jax.experimental.pallas import tpu as pltpu
from jax.experimental.pallas import tpu_sc as plsc

_N = 2048
_DEG = 16
_EPS = 1e-5
_RB = 256
_KS = _N // _RB
_NC, _NS = 2, 16
_NW = _NC * _NS
_NPW = _N // _NW


def _pad128(d):
    return ((d + 127) // 128) * 128


@functools.lru_cache(maxsize=None)
def _make_conv(d, h1, h2):
    f32 = jnp.float32
    P = _pad128(d)

    def body(A_ref, X_ref,
             W1a_ref, b1a_ref, W2a_ref, b2a_ref, ga_ref, bta_ref,
             W1b_ref, b1b_ref, W2b_ref, b2b_ref, gb_ref, btb_ref,
             y1_ref, y2_ref,
             y1s, y2s, s1, q1, s2, q2):
        j = pl.program_id(0)

        @pl.when(j == 0)
        def _():
            s1[...] = jnp.zeros_like(s1)
            q1[...] = jnp.zeros_like(q1)
            s2[...] = jnp.zeros_like(s2)
            q2[...] = jnp.zeros_like(q2)

        X = X_ref[...]
        agg = jnp.dot(A_ref[...], X, preferred_element_type=f32)
        t1 = jnp.maximum(
            jnp.dot(agg, W1a_ref[...], preferred_element_type=f32) + b1a_ref[...], 0.0)
        r1 = jnp.dot(t1, W2a_ref[...], preferred_element_type=f32) + b2a_ref[...]
        t2 = jnp.maximum(
            jnp.dot(agg, W1b_ref[...], preferred_element_type=f32) + b1b_ref[...], 0.0)
        r2 = jnp.dot(t2, W2b_ref[...], preferred_element_type=f32) + b2b_ref[...]
        y1s[pl.ds(j * _RB, _RB), :] = r1
        y2s[pl.ds(j * _RB, _RB), :] = r2
        s1[...] += jnp.sum(r1, axis=0, keepdims=True)
        q1[...] += jnp.sum(r1 * r1, axis=0, keepdims=True)
        s2[...] += jnp.sum(r2, axis=0, keepdims=True)
        q2[...] += jnp.sum(r2 * r2, axis=0, keepdims=True)

        @pl.when(j == _KS - 1)
        def _():
            inv_n = 1.0 / _N
            m1 = s1[...] * inv_n
            v1 = q1[...] * inv_n - m1 * m1
            sc1 = ga_ref[...] * lax.rsqrt(v1 + _EPS)
            y1_ref[...] = jnp.maximum((y1s[...] - m1) * sc1 + bta_ref[...], 0.0)
            m2 = s2[...] * inv_n
            v2 = q2[...] * inv_n - m2 * m2
            sc2 = gb_ref[...] * lax.rsqrt(v2 + _EPS)
            y2_ref[...] = jnp.maximum((y2s[...] - m2) * sc2 + btb_ref[...], 0.0)

    def whole(shape):
        nd = len(shape)
        return pl.BlockSpec(shape, lambda j, _nd=nd: (0,) * _nd)

    return pl.pallas_call(
        body,
        grid=(_KS,),
        in_specs=[
            pl.BlockSpec((_RB, _N), lambda j: (j, 0)),
            whole((_N, d)),
            whole((d, h1)), whole((1, h1)), whole((h1, 128)), whole((1, 128)),
            whole((1, 128)), whole((1, 128)),
            whole((d, h2)), whole((1, h2)), whole((h2, 64)), whole((1, 64)),
            whole((1, 64)), whole((1, 64)),
        ],
        out_specs=[whole((_N, 128)), whole((_N, 64))],
        out_shape=[
            jax.ShapeDtypeStruct((_N, 128), f32),
            jax.ShapeDtypeStruct((_N, 64), f32),
        ],
        scratch_shapes=[
            pltpu.VMEM((_N, 128), f32), pltpu.VMEM((_N, 64), f32),
            pltpu.VMEM((1, 128), f32), pltpu.VMEM((1, 128), f32),
            pltpu.VMEM((1, 64), f32), pltpu.VMEM((1, 64), f32),
        ],
        compiler_params=pltpu.CompilerParams(
            dimension_semantics=("arbitrary",)),
    )


@functools.lru_cache(maxsize=None)
def _make_tablize(d):
    f32 = jnp.float32
    P = _pad128(d)

    def body(X_ref, table_ref):
        X = X_ref[...]
        cmin = jnp.min(X, axis=0, keepdims=True)
        rows = jnp.concatenate([X, cmin], axis=0)
        table_ref[...] = jnp.concatenate(
            [rows, jnp.zeros((_N + 1, P - d), f32)], axis=1)

    return pl.pallas_call(
        body,
        out_shape=jax.ShapeDtypeStruct((_N + 1, P), f32),
    )


@functools.lru_cache(maxsize=None)
def _make_maxpool(d):
    f32 = jnp.float32
    P = _pad128(d)
    npc = 8 if (_DEG * P * 4 * 2 * 8) <= 420000 else 4
    gs = npc * _DEG
    nch = _NPW // npc
    cpw = 8 // npc
    mesh = plsc.VectorSubcoreMesh(core_axis_name="c", subcore_axis_name="s")

    @functools.partial(
        pl.kernel,
        mesh=mesh,
        out_type=jax.ShapeDtypeStruct((_N, P), f32),
        scratch_types=[
            pltpu.VMEM((gs, P), f32),
            pltpu.VMEM((gs, P), f32),
            pltpu.VMEM((gs,), jnp.int32),
            pltpu.VMEM((gs,), jnp.int32),
            pltpu.VMEM((8, P), f32),
            pltpu.SemaphoreType.DMA,
            pltpu.SemaphoreType.DMA,
        ],
    )
    def mp(table_hbm, pnl_hbm, out_hbm, rows0, rows1, idx0, idx1, obuf,
           sem0, sem1):
        wid = lax.axis_index("s") * _NC + lax.axis_index("c")
        node0 = wid * _NPW
        rows = (rows0, rows1)
        idxs = (idx0, idx1)
        sems = (sem0, sem1)

        def start(g, b):
            pltpu.sync_copy(
                pnl_hbm.at[pl.ds((node0 + g * npc) * _DEG, gs)], idxs[b])
            pltpu.async_copy(table_hbm.at[idxs[b]], rows[b], sems[b])

        start(0, 0)
        for g in range(nch):
            b = g % 2
            pltpu.make_async_copy(table_hbm.at[idxs[b]], rows[b],
                                  sems[b]).wait()
            if g + 1 < nch:
                start(g + 1, 1 - b)
            r = rows[b]
            orow = (g % cpw) * npc
            for p in range(npc):
                def col(ci, carry, _p=p, _orow=orow):
                    c0 = ci * 16
                    a = r[_p * _DEG, pl.ds(c0, 16)]
                    for k in range(1, _DEG):
                        a = jnp.maximum(a, r[_p * _DEG + k, pl.ds(c0, 16)])
                    obuf[_orow + _p, pl.ds(c0, 16)] = a
                    return carry
                lax.fori_loop(0, P // 16, col, 0)
            if (g + 1) % cpw == 0:
                pltpu.sync_copy(
                    obuf, out_hbm.at[pl.ds(node0 + (g + 1 - cpw) * npc, 8)])

    return mp


@functools.lru_cache(maxsize=None)
def _make_head(d):
    f32 = jnp.float32

    def body(X_ref, W1_ref, b1_ref, al_ref, W2_ref, b2_ref, out_ref):
        pooled = jnp.sum(X_ref[...], axis=0, keepdims=True) * (1.0 / _N)
        z = jnp.dot(pooled, W1_ref[...], preferred_element_type=f32) + b1_ref[...]
        z = jnp.where(z > 0.0, z, al_ref[...] * z)
        r = jnp.dot(z, W2_ref[...], preferred_element_type=f32) + b2_ref[...]
        rp = jnp.concatenate([r, jnp.zeros((1, 126), f32)], axis=1)
        out_ref[...] = jnp.concatenate([rp, jnp.zeros((7, 128), f32)], axis=0)

    return pl.pallas_call(
        body,
        out_shape=jax.ShapeDtypeStruct((8, 128), f32),
    )


def kernel(h, A, padded_neighbor_list,
           gc1_W1_0, gc1_b1_0, gc1_W2_0, gc1_b2_0, gc1_gamma_0, gc1_beta_0,
           gc2_W1_0, gc2_b1_0, gc2_W2_0, gc2_b2_0, gc2_gamma_0, gc2_beta_0,
           gc1_W1_1, gc1_b1_1, gc1_W2_1, gc1_b2_1, gc1_gamma_1, gc1_beta_1,
           gc2_W1_1, gc2_b1_1, gc2_W2_1, gc2_b2_1, gc2_gamma_1, gc2_beta_1,
           gc1_W1_2, gc1_b1_2, gc1_W2_2, gc1_b2_2, gc1_gamma_2, gc1_beta_2,
           gc2_W1_2, gc2_b1_2, gc2_W2_2, gc2_b2_2, gc2_gamma_2, gc2_beta_2,
           gc1_W1_3, gc1_b1_3, gc1_W2_3, gc1_b2_3, gc1_gamma_3, gc1_beta_3,
           gc2_W1_3, gc2_b1_3, gc2_W2_3, gc2_b2_3, gc2_gamma_3, gc2_beta_3,
           gc1_W1_4, gc1_b1_4, gc1_W2_4, gc1_b2_4, gc1_gamma_4, gc1_beta_4,
           gc2_W1_4, gc2_b1_4, gc2_W2_4, gc2_b2_4, gc2_gamma_4, gc2_beta_4,
           Wc1, bc1, alpha, Wc2, bc2):
    prm = dict(locals())
    f32 = jnp.float32
    X = h[0].astype(f32)
    pnl_flat = padded_neighbor_list.astype(jnp.int32).reshape(-1)

    for i in range(5):
        d = X.shape[1]
        W1a = prm['gc1_W1_%d' % i]; h1 = W1a.shape[1]
        W1b = prm['gc2_W1_%d' % i]; h2 = W1b.shape[1]
        table = _make_tablize(d)(X)
        conv = _make_conv(d, h1, h2)
        y1, y2 = conv(
            A, X,
            W1a, prm['gc1_b1_%d' % i].reshape(1, -1),
            prm['gc1_W2_%d' % i], prm['gc1_b2_%d' % i].reshape(1, -1),
            prm['gc1_gamma_%d' % i].reshape(1, -1),
            prm['gc1_beta_%d' % i].reshape(1, -1),
            W1b, prm['gc2_b1_%d' % i].reshape(1, -1),
            prm['gc2_W2_%d' % i], prm['gc2_b2_%d' % i].reshape(1, -1),
            prm['gc2_gamma_%d' % i].reshape(1, -1),
            prm['gc2_beta_%d' % i].reshape(1, -1),
        )
        o1 = _make_maxpool(d)(table, pnl_flat)
        X = jnp.concatenate([o1[:, :d], y1, y2], axis=1)

    head = _make_head(X.shape[1])
    res = head(X, Wc1, bc1.reshape(1, -1), alpha.reshape(1, -1),
               Wc2, bc2.reshape(1, -1))
    return res[0:1, 0:2]
